# initial kernel scaffold (unmeasured)
import jax
import jax.numpy as jnp
from jax import lax
from jax.experimental import pallas as pl
from jax.experimental.pallas import tpu as pltpu

N_DEV = 4
M = 1024
K = 4096
NCOL = 2048
SUB = 512
SUBS_PER_DEST = NCOL // SUB
T = N_DEV * SUBS_PER_DEST

_MESH = pl.DeviceIdType.MESH


def kernel(x, w_mat):
    x = x.astype(jnp.bfloat16)

    def body(x_ref, w_ref, out_ref, wbuf, ybuf, load_sems, send_sems, recv_sems):
        my = lax.axis_index("i")

        barrier = pltpu.get_barrier_semaphore()
        for k in (1, 2, 3):
            pl.semaphore_signal(
                barrier, inc=1,
                device_id=((my + k) % N_DEV,), device_id_type=_MESH,
            )
        pl.semaphore_wait(barrier, 3)

        def w_col(t):
            d = (my + 1 + t // SUBS_PER_DEST) % N_DEV
            return d * NCOL + (t % SUBS_PER_DEST) * SUB

        def load(t):
            return pltpu.make_async_copy(
                w_ref.at[:, pl.ds(w_col(t), SUB)],
                wbuf.at[t % 2],
                load_sems.at[t % 2],
            )

        load(0).start()
        rdmas = []
        for t in range(T):
            load(t).wait()
            if t + 1 < T:
                load(t + 1).start()
            wb = wbuf[t % 2].astype(jnp.bfloat16)
            y = jnp.dot(
                x_ref[:, :], wb, preferred_element_type=jnp.float32
            ).astype(jnp.bfloat16)
            g = t // SUBS_PER_DEST
            c = (t % SUBS_PER_DEST) * SUB
            if g < 3:
                ybuf[g, :, pl.ds(c, SUB)] = y
                if t % SUBS_PER_DEST == SUBS_PER_DEST - 1:
                    rdma = pltpu.make_async_remote_copy(
                        src_ref=ybuf.at[g],
                        dst_ref=out_ref.at[pl.ds(my * M, M), :],
                        send_sem=send_sems.at[g],
                        recv_sem=recv_sems.at[g],
                        device_id=((my + 1 + g) % N_DEV,),
                        device_id_type=_MESH,
                    )
                    rdma.start()
                    rdmas.append(rdma)
            else:
                out_ref[pl.ds(my * M, M), pl.ds(c, SUB)] = y

        for s in range(3):
            origin = (my - 1 - s) % N_DEV
            pltpu.make_async_remote_copy(
                src_ref=ybuf.at[s],
                dst_ref=out_ref.at[pl.ds(origin * M, M), :],
                send_sem=send_sems.at[s],
                recv_sem=recv_sems.at[s],
                device_id=(my,),
                device_id_type=_MESH,
            ).wait_recv()
        for r in rdmas:
            r.wait_send()

    return pl.pallas_call(
        body,
        out_shape=jax.ShapeDtypeStruct((N_DEV * M, NCOL), jnp.bfloat16),
        in_specs=[
            pl.BlockSpec(memory_space=pltpu.VMEM),
            pl.BlockSpec(memory_space=pl.ANY),
        ],
        out_specs=pl.BlockSpec(memory_space=pltpu.VMEM),
        scratch_shapes=[
            pltpu.VMEM((2, K, SUB), jnp.float32),
            pltpu.VMEM((3, M, NCOL), jnp.bfloat16),
            pltpu.SemaphoreType.DMA((2,)),
            pltpu.SemaphoreType.DMA((3,)),
            pltpu.SemaphoreType.DMA((3,)),
        ],
        compiler_params=pltpu.CompilerParams(collective_id=0),
    )(x, w_mat)


# baseline (device time: 196673 ns/iter reference)
import jax

jax.config.update("jax_compilation_cache_dir", "/tmp/jax_pallas_cache")
jax.config.update("jax_persistent_cache_min_compile_time_secs", 0)

import jax.numpy as jnp
from jax import lax
from jax.experimental import pallas as pl
from jax.experimental.pallas import tpu as pltpu

N_DEV = 4
M = 1024
K = 4096
NCOL = 2048
SUB = 256
SUBS_PER_DEST = NCOL // SUB
T = N_DEV * SUBS_PER_DEST

_MESH = pl.DeviceIdType.MESH


def kernel(x, w_mat):
    x = x.astype(jnp.bfloat16)

    def body(x_ref, w_ref, out_ref, wbuf, ybuf, load_sems, send_sems, recv_sems):
        my = lax.axis_index("i")

        barrier = pltpu.get_barrier_semaphore()
        for k in (1, 2, 3):
            pl.semaphore_signal(
                barrier, inc=1,
                device_id=((my + k) % N_DEV,), device_id_type=_MESH,
            )
        pl.semaphore_wait(barrier, 3)

        def w_col(t):
            d = (my + 1 + t // SUBS_PER_DEST) % N_DEV
            return d * NCOL + (t % SUBS_PER_DEST) * SUB

        def load(t):
            return pltpu.make_async_copy(
                w_ref.at[:, pl.ds(w_col(t), SUB)],
                wbuf.at[t % 2],
                load_sems.at[t % 2],
            )

        load(0).start()
        rdmas = []
        for t in range(T):
            load(t).wait()
            if t + 1 < T:
                load(t + 1).start()
            wb = wbuf[t % 2].astype(jnp.bfloat16)
            y = jnp.dot(
                x_ref[:, :], wb, preferred_element_type=jnp.float32
            ).astype(jnp.bfloat16)
            g = t // SUBS_PER_DEST
            c = (t % SUBS_PER_DEST) * SUB
            if g < 3:
                ybuf[g, :, pl.ds(c, SUB)] = y
                if t % SUBS_PER_DEST == SUBS_PER_DEST - 1:
                    rdma = pltpu.make_async_remote_copy(
                        src_ref=ybuf.at[g],
                        dst_ref=out_ref.at[pl.ds(my * M, M), :],
                        send_sem=send_sems.at[g],
                        recv_sem=recv_sems.at[g],
                        device_id=((my + 1 + g) % N_DEV,),
                        device_id_type=_MESH,
                    )
                    rdma.start()
                    rdmas.append(rdma)
            else:
                out_ref[pl.ds(my * M, M), pl.ds(c, SUB)] = y

        for s in range(3):
            origin = (my - 1 - s) % N_DEV
            pltpu.make_async_remote_copy(
                src_ref=ybuf.at[s],
                dst_ref=out_ref.at[pl.ds(origin * M, M), :],
                send_sem=send_sems.at[s],
                recv_sem=recv_sems.at[s],
                device_id=(my,),
                device_id_type=_MESH,
            ).wait_recv()
        for r in rdmas:
            r.wait_send()

    return pl.pallas_call(
        body,
        out_shape=jax.ShapeDtypeStruct((N_DEV * M, NCOL), jnp.bfloat16),
        in_specs=[
            pl.BlockSpec(memory_space=pltpu.VMEM),
            pl.BlockSpec(memory_space=pl.ANY),
        ],
        out_specs=pl.BlockSpec(memory_space=pltpu.VMEM),
        scratch_shapes=[
            pltpu.VMEM((2, K, SUB), jnp.float32),
            pltpu.VMEM((3, M, NCOL), jnp.bfloat16),
            pltpu.SemaphoreType.DMA((2,)),
            pltpu.SemaphoreType.DMA((3,)),
            pltpu.SemaphoreType.DMA((3,)),
        ],
        compiler_params=pltpu.CompilerParams(
            collective_id=0,
            vmem_limit_bytes=40 * 1024 * 1024,
        ),
    )(x, w_mat)


# device time: 173413 ns/iter; 1.1341x vs baseline; 1.1341x over previous
import jax

jax.config.update("jax_compilation_cache_dir", "/tmp/jax_pallas_cache")
jax.config.update("jax_persistent_cache_min_compile_time_secs", 0)

import jax.numpy as jnp
from jax import lax
from jax.experimental import pallas as pl
from jax.experimental.pallas import tpu as pltpu

N_DEV = 4
M = 1024
K = 4096
NCOL = 2048
SUB = 512
SPG = NCOL // SUB
T = N_DEV * SPG
HALF = 1024

_MESH = pl.DeviceIdType.MESH


def kernel(x, w_mat):
    x = x.astype(jnp.bfloat16)

    def body(x_ref, w_ref, out_ref, wbuf, ybuf,
             load_sems, send_sems, recv_sems, local_sems):
        my = lax.axis_index("i")
        even = my % 2 == 0

        barrier = pltpu.get_barrier_semaphore()
        for k in (1, 2, 3):
            pl.semaphore_signal(
                barrier, inc=1,
                device_id=((my + k) % N_DEV,), device_id_type=_MESH,
            )
        pl.semaphore_wait(barrier, 3)

        def dest(g):
            if g == 3:
                return my
            off = jnp.where(even, 3 - g, g + 1)
            return (my + off) % N_DEV

        def w_col(t):
            return dest(t // SPG) * NCOL + (t % SPG) * SUB

        def load(t):
            return pltpu.make_async_copy(
                w_ref.at[:, pl.ds(w_col(t), SUB)],
                wbuf.at[t % 2],
                load_sems.at[t % 2],
            )

        load(0).start()
        rdmas = []
        for t in range(T):
            load(t).wait()
            if t + 1 < T:
                load(t + 1).start()
            wb = wbuf[t % 2].astype(jnp.bfloat16)
            y = jnp.dot(
                x_ref[:, :], wb, preferred_element_type=jnp.float32
            ).astype(jnp.bfloat16)
            g = t // SPG
            h = (t % SPG) // 2
            slot = 2 * g + h
            ybuf[slot, :, pl.ds((t % 2) * SUB, SUB)] = y
            if t % 2 == 1:
                if g < 3:
                    rdma = pltpu.make_async_remote_copy(
                        src_ref=ybuf.at[slot],
                        dst_ref=out_ref.at[pl.ds(my * M, M),
                                           pl.ds(h * HALF, HALF)],
                        send_sem=send_sems.at[slot],
                        recv_sem=recv_sems.at[slot],
                        device_id=(dest(g),),
                        device_id_type=_MESH,
                    )
                    rdma.start()
                    rdmas.append(rdma)
                else:
                    pltpu.make_async_copy(
                        ybuf.at[slot],
                        out_ref.at[pl.ds(my * M, M), pl.ds(h * HALF, HALF)],
                        local_sems.at[h],
                    ).start()

        for slot in range(6):
            pltpu.make_async_remote_copy(
                src_ref=ybuf.at[slot],
                dst_ref=out_ref.at[pl.ds(my * M, M), pl.ds(0, HALF)],
                send_sem=send_sems.at[slot],
                recv_sem=recv_sems.at[slot],
                device_id=(my,),
                device_id_type=_MESH,
            ).wait_recv()
        for r in rdmas:
            r.wait_send()
        for h in range(2):
            pltpu.make_async_copy(
                ybuf.at[6 + h],
                out_ref.at[pl.ds(my * M, M), pl.ds(h * HALF, HALF)],
                local_sems.at[h],
            ).wait()

    return pl.pallas_call(
        body,
        out_shape=jax.ShapeDtypeStruct((N_DEV * M, NCOL), jnp.bfloat16),
        in_specs=[
            pl.BlockSpec(memory_space=pltpu.VMEM),
            pl.BlockSpec(memory_space=pl.ANY),
        ],
        out_specs=pl.BlockSpec(memory_space=pl.ANY),
        scratch_shapes=[
            pltpu.VMEM((2, K, SUB), jnp.float32),
            pltpu.VMEM((8, M, HALF), jnp.bfloat16),
            pltpu.SemaphoreType.DMA((2,)),
            pltpu.SemaphoreType.DMA((6,)),
            pltpu.SemaphoreType.DMA((6,)),
            pltpu.SemaphoreType.DMA((2,)),
        ],
        compiler_params=pltpu.CompilerParams(
            collective_id=0,
            vmem_limit_bytes=48 * 1024 * 1024,
        ),
    )(x, w_mat)


# device time: 171859 ns/iter; 1.1444x vs baseline; 1.0090x over previous
import jax

jax.config.update("jax_compilation_cache_dir", "/tmp/jax_pallas_cache")
jax.config.update("jax_persistent_cache_min_compile_time_secs", 0)

import jax.numpy as jnp
from jax import lax
from jax.experimental import pallas as pl
from jax.experimental.pallas import tpu as pltpu

N_DEV = 4
M = 1024
K = 4096
NCOL = 2048
SUB = 512
SPG = NCOL // SUB
T = N_DEV * SPG
HALF = 1024

_MESH = pl.DeviceIdType.MESH


def kernel(x, w_mat):
    x = x.astype(jnp.bfloat16)

    def body(x_ref, w_ref, out_ref, wbuf, ybuf,
             load_sems, send_sems, recv_sems, local_sems):
        my = lax.axis_index("i")
        even = my % 2 == 0

        barrier = pltpu.get_barrier_semaphore()
        for k in (1, 2, 3):
            pl.semaphore_signal(
                barrier, inc=1,
                device_id=((my + k) % N_DEV,), device_id_type=_MESH,
            )

        def dest(g):
            if g == 3:
                return my
            off = jnp.where(even, 3 - g, g + 1)
            return (my + off) % N_DEV

        def w_col(t):
            return dest(t // SPG) * NCOL + (t % SPG) * SUB

        def load(t):
            return pltpu.make_async_copy(
                w_ref.at[:, pl.ds(w_col(t), SUB)],
                wbuf.at[t % 2],
                load_sems.at[t % 2],
            )

        load(0).start()
        rdmas = []
        for t in range(T):
            load(t).wait()
            if t + 1 < T:
                load(t + 1).start()
            wb = wbuf[t % 2].astype(jnp.bfloat16)
            y = jnp.dot(
                x_ref[:, :], wb, preferred_element_type=jnp.float32
            ).astype(jnp.bfloat16)
            g = t // SPG
            h = (t % SPG) // 2
            slot = 2 * g + h
            ybuf[slot, :, pl.ds((t % 2) * SUB, SUB)] = y
            if t % 2 == 1:
                if t == 1:
                    pl.semaphore_wait(barrier, 3)
                if g < 3:
                    rdma = pltpu.make_async_remote_copy(
                        src_ref=ybuf.at[slot],
                        dst_ref=out_ref.at[pl.ds(my * M, M),
                                           pl.ds(h * HALF, HALF)],
                        send_sem=send_sems.at[slot],
                        recv_sem=recv_sems.at[slot],
                        device_id=(dest(g),),
                        device_id_type=_MESH,
                    )
                    rdma.start()
                    rdmas.append(rdma)
                else:
                    pltpu.make_async_copy(
                        ybuf.at[slot],
                        out_ref.at[pl.ds(my * M, M), pl.ds(h * HALF, HALF)],
                        local_sems.at[h],
                    ).start()

        for slot in range(6):
            pltpu.make_async_remote_copy(
                src_ref=ybuf.at[slot],
                dst_ref=out_ref.at[pl.ds(my * M, M), pl.ds(0, HALF)],
                send_sem=send_sems.at[slot],
                recv_sem=recv_sems.at[slot],
                device_id=(my,),
                device_id_type=_MESH,
            ).wait_recv()
        for r in rdmas:
            r.wait_send()
        for h in range(2):
            pltpu.make_async_copy(
                ybuf.at[6 + h],
                out_ref.at[pl.ds(my * M, M), pl.ds(h * HALF, HALF)],
                local_sems.at[h],
            ).wait()

    return pl.pallas_call(
        body,
        out_shape=jax.ShapeDtypeStruct((N_DEV * M, NCOL), jnp.bfloat16),
        in_specs=[
            pl.BlockSpec(memory_space=pltpu.VMEM),
            pl.BlockSpec(memory_space=pl.ANY),
        ],
        out_specs=pl.BlockSpec(memory_space=pl.ANY),
        scratch_shapes=[
            pltpu.VMEM((2, K, SUB), jnp.float32),
            pltpu.VMEM((8, M, HALF), jnp.bfloat16),
            pltpu.SemaphoreType.DMA((2,)),
            pltpu.SemaphoreType.DMA((6,)),
            pltpu.SemaphoreType.DMA((6,)),
            pltpu.SemaphoreType.DMA((2,)),
        ],
        compiler_params=pltpu.CompilerParams(
            collective_id=0,
            vmem_limit_bytes=48 * 1024 * 1024,
        ),
    )(x, w_mat)
